# Initial kernel scaffold; baseline (speedup 1.0000x reference)
import jax
import jax.numpy as jnp
from jax.experimental import pallas as pl


def kernel(anchors, objectness, box_regression):
    raise NotImplementedError("write your pallas kernel here")


# speedup vs baseline: 17.2946x; 17.2946x over previous
import math

import jax
import jax.numpy as jnp
from jax.experimental import pallas as pl

PRE_NMS_TOP_N = 2000
FPN_POST_NMS_TOP_N = 1000
NMS_THRESH = 0.7
IMG_H = 1024.0
IMG_W = 1024.0
BBOX_XFORM_CLIP = math.log(1000.0 / 16.0)
PAD_N = 2048


def _decode_kernel(sl_ref, dx_ref, dy_ref, dw_ref, dh_ref,
                   ax1_ref, ay1_ref, ax2_ref, ay2_ref,
                   bx1_ref, by1_ref, bx2_ref, by2_ref, sm_ref, v_ref):
    col = jax.lax.broadcasted_iota(jnp.int32, sl_ref.shape, 1)
    real = col < PRE_NMS_TOP_N

    ax1 = ax1_ref[...]
    ay1 = ay1_ref[...]
    w = ax2_ref[...] - ax1 + 1.0
    h = ay2_ref[...] - ay1 + 1.0
    cx = ax1 + 0.5 * w
    cy = ay1 + 0.5 * h
    dw = jnp.minimum(dw_ref[...], BBOX_XFORM_CLIP)
    dh = jnp.minimum(dh_ref[...], BBOX_XFORM_CLIP)
    pcx = dx_ref[...] * w + cx
    pcy = dy_ref[...] * h + cy
    pw = jnp.exp(dw) * w
    ph = jnp.exp(dh) * h

    x1 = jnp.clip(pcx - 0.5 * pw, 0.0, IMG_W - 1.0)
    y1 = jnp.clip(pcy - 0.5 * ph, 0.0, IMG_H - 1.0)
    x2 = jnp.clip(pcx + 0.5 * pw - 1.0, 0.0, IMG_W - 1.0)
    y2 = jnp.clip(pcy + 0.5 * ph - 1.0, 0.0, IMG_H - 1.0)

    ws = x2 - x1 + 1.0
    hs = y2 - y1 + 1.0
    xc = x1 + ws / 2.0
    yc = y1 + hs / 2.0
    valid = (ws >= 0.0) & (hs >= 0.0) & (xc < IMG_W) & (yc < IMG_H) & real

    s = jax.nn.sigmoid(sl_ref[...])
    sm_ref[...] = jnp.where(valid, s, -1e4)
    v_ref[...] = jnp.where(valid, 1.0, 0.0)
    bx1_ref[...] = jnp.where(real, x1, 0.0)
    by1_ref[...] = jnp.where(real, y1, 0.0)
    bx2_ref[...] = jnp.where(real, x2, -1.0)
    by2_ref[...] = jnp.where(real, y2, -1.0)


def _nms_kernel(x1_ref, y1_ref, x2_ref, y2_ref, v_ref, s_ref, out_ref):
    x1 = x1_ref[...]
    y1 = y1_ref[...]
    x2 = x2_ref[...]
    y2 = y2_ref[...]
    area = (x2 - x1 + 1.0) * (y2 - y1 + 1.0)
    col = jax.lax.broadcasted_iota(jnp.int32, x1.shape, 1)

    def _pivot(a, mask):
        return jnp.sum(jnp.where(mask, a, 0.0), axis=1, keepdims=True)

    def body(i, keep):
        pm = col == i
        px1 = _pivot(x1, pm)
        py1 = _pivot(y1, pm)
        px2 = _pivot(x2, pm)
        py2 = _pivot(y2, pm)
        pk = _pivot(keep, pm)
        pa = (px2 - px1 + 1.0) * (py2 - py1 + 1.0)
        iw = jnp.maximum(jnp.minimum(x2, px2) - jnp.maximum(x1, px1) + 1.0, 0.0)
        ih = jnp.maximum(jnp.minimum(y2, py2) - jnp.maximum(y1, py1) + 1.0, 0.0)
        inter = iw * ih
        iou = inter / (area + pa - inter)
        suppress = (iou > NMS_THRESH) & (col > i) & (pk > 0.5)
        return jnp.where(suppress, 0.0, keep)

    keep = jax.lax.fori_loop(
        0, PRE_NMS_TOP_N, body, jnp.ones(x1.shape, jnp.float32))
    out_ref[...] = jnp.where(
        (keep > 0.5) & (v_ref[...] > 0.5), s_ref[...], -1e4)


def _row_call(fn, n_out, inputs):
    n = inputs[0].shape[0]
    return pl.pallas_call(
        fn,
        out_shape=[jax.ShapeDtypeStruct((n, PAD_N), jnp.float32)] * n_out,
    )(*inputs)


def _pad(x):
    return jnp.pad(x, ((0, 0), (0, PAD_N - x.shape[1])))


@jax.jit
def _kernel_impl(anchors, objectness, box_regression):
    N, A, H, W = objectness.shape
    obj = jnp.transpose(objectness, (0, 2, 3, 1)).reshape(N, -1)
    reg = jnp.transpose(
        box_regression.reshape(N, -1, 4, H, W), (0, 3, 4, 1, 2)).reshape(N, -1, 4)

    s_logit, idx = jax.lax.top_k(obj, PRE_NMS_TOP_N)
    reg_t = jnp.take_along_axis(reg, idx[..., None], axis=1)
    anch_t = anchors[idx]

    dec_in = [s_logit,
              reg_t[..., 0], reg_t[..., 1], reg_t[..., 2], reg_t[..., 3],
              anch_t[..., 0], anch_t[..., 1], anch_t[..., 2], anch_t[..., 3]]
    dec_in = [_pad(x) for x in dec_in]
    bx1, by1, bx2, by2, s_m, valid = _row_call(_decode_kernel, 6, dec_in)

    order = jnp.argsort(-s_m, axis=1)
    x1o, y1o, x2o, y2o, vo, so = (
        jnp.take_along_axis(a, order, axis=1)
        for a in (bx1, by1, bx2, by2, valid, s_m))

    s_final = _row_call(_nms_kernel, 1, [x1o, y1o, x2o, y2o, vo, so])[0]

    top_s, top_i = jax.lax.top_k(s_final[:, :PRE_NMS_TOP_N], FPN_POST_NMS_TOP_N)
    boxes = jnp.stack(
        [jnp.take_along_axis(a, top_i, axis=1) for a in (x1o, y1o, x2o, y2o)],
        axis=-1)
    return jnp.concatenate([boxes, top_s[..., None]], axis=-1)


def kernel(anchors, objectness, box_regression):
    return _kernel_impl(anchors, objectness, box_regression)
